# Initial kernel scaffold; baseline (speedup 1.0000x reference)
import functools

import jax
import jax.numpy as jnp
from jax import lax
from jax.experimental import pallas as pl
from jax.experimental.pallas import tpu as pltpu
from jax.experimental.pallas import tpu_sc as plsc

N = 10000
E = 320000
D_IN = 128
HID = 16

NC = 2
NS = 16
NW = NC * NS
L = 16

BLK = 128
NBLK = 80
EPT = NBLK * BLK
E_PAD = NW * EPT
ROWS_PT = N // NS

_f32 = jnp.float32


def _dense_in_body(x_ref, w_ref, asrc_ref, adst_ref, xw_out, as_out, ad_out):
  xw = jnp.dot(x_ref[:], w_ref[:], preferred_element_type=_f32)
  xw_out[:] = xw
  as_out[:] = jnp.dot(xw, asrc_ref[:], preferred_element_type=_f32)
  ad_out[:] = jnp.dot(xw, adst_ref[:], preferred_element_type=_f32)


def _dense_in(x, w, asrc, adst):
  return pl.pallas_call(
      _dense_in_body,
      out_shape=[
          jax.ShapeDtypeStruct((N, HID), _f32),
          jax.ShapeDtypeStruct((N, 1), _f32),
          jax.ShapeDtypeStruct((N, 1), _f32),
      ],
  )(x, w, asrc, adst)


def _dense_mid_body(nump_ref, denp_ref, b_ref, w_ref, asrc_ref, adst_ref,
                    xw_out, as_out, ad_out):
  num = jnp.sum(nump_ref[:], axis=0)
  den = jnp.sum(denp_ref[:], axis=0)
  x = jnp.maximum(num / (den[:, None] + 1e-16) + b_ref[:], 0.0)
  xw = jnp.dot(x, w_ref[:], preferred_element_type=_f32)
  xw_out[:] = xw
  as_out[:] = jnp.dot(xw, asrc_ref[:], preferred_element_type=_f32)
  ad_out[:] = jnp.dot(xw, adst_ref[:], preferred_element_type=_f32)


def _dense_mid(nump, denp, b, w, asrc, adst):
  return pl.pallas_call(
      _dense_mid_body,
      out_shape=[
          jax.ShapeDtypeStruct((N, HID), _f32),
          jax.ShapeDtypeStruct((N, 1), _f32),
          jax.ShapeDtypeStruct((N, 1), _f32),
      ],
  )(nump, denp, b, w, asrc, adst)


def _dense_out_body(nump_ref, denp_ref, b_ref, w_ref, bout_ref, out_ref):
  num = jnp.sum(nump_ref[:], axis=0)
  den = jnp.sum(denp_ref[:], axis=0)
  x = jnp.maximum(num / (den[:, None] + 1e-16) + b_ref[:], 0.0)
  out_ref[:] = jnp.dot(x, w_ref[:], preferred_element_type=_f32) + bout_ref[:]


def _dense_out(nump, denp, b, w, bout):
  return pl.pallas_call(
      _dense_out_body,
      out_shape=jax.ShapeDtypeStruct((N, 1), _f32),
  )(nump, denp, b, w, bout)


def _edge_body(src_hbm, dst_hbm, asrc_hbm, adst_hbm, xw_hbm,
               num_out, den_out,
               asrc_v, adst_v, den_v, src_v, dst_v, rows_v, ex_v,
               num_sh, gsem, ssem):
  cid = lax.axis_index("c")
  sid = lax.axis_index("s")
  wid = sid * NC + cid

  pltpu.sync_copy(src_hbm.at[wid], src_v)
  pltpu.sync_copy(dst_hbm.at[wid], dst_v)
  pltpu.sync_copy(asrc_hbm, asrc_v)
  pltpu.sync_copy(adst_hbm, adst_v)

  zero16 = jnp.zeros((L,), _f32)

  def _zden(i, carry):
    den_v[pl.ds(i * L, L)] = zero16
    return carry
  lax.fori_loop(0, N // L, _zden, 0)

  def _zrow(j, carry):
    rows_v[j] = zero16
    return carry
  lax.fori_loop(0, BLK, _zrow, 0)
  row0 = sid * ROWS_PT
  for r in range(0, ROWS_PT, BLK):
    n = min(BLK, ROWS_PT - r)
    pltpu.sync_copy(rows_v.at[pl.ds(0, n)], num_sh.at[pl.ds(row0 + r, n)])
  plsc.subcore_barrier()

  iota = lax.iota(jnp.int32, L)
  ebase = wid * EPT

  def _block(blk, carry):
    gather = pltpu.async_copy(xw_hbm.at[src_v.at[blk]], rows_v, gsem)

    def _chunk(j, c2):
      s_vec = src_v[blk, pl.ds(j * L, L)]
      d_vec = dst_v[blk, pl.ds(j * L, L)]
      a = plsc.load_gather(asrc_v, [s_vec]) + plsc.load_gather(adst_v, [d_vec])
      a = jnp.maximum(a, a * 0.2)
      ex = jnp.exp(a)
      eid = ebase + blk * BLK + j * L + iota
      ex = jnp.where(eid < E, ex, 0.0)
      plsc.addupdate_scatter(den_v, [d_vec], ex)
      ex_v[pl.ds(j * L, L)] = ex
      return c2
    lax.fori_loop(0, BLK // L, _chunk, 0)

    gather.wait()

    def _scale(j, c2):
      rows_v[j] = rows_v[j] * ex_v[j]
      return c2
    lax.fori_loop(0, BLK, _scale, 0)

    pltpu.async_copy(rows_v, num_sh.at[dst_v.at[blk]], ssem, add=True).wait()
    return carry

  lax.fori_loop(0, NBLK, _block, 0)
  plsc.subcore_barrier()

  pltpu.sync_copy(den_v, den_out.at[wid])
  pltpu.sync_copy(num_sh.at[pl.ds(row0, ROWS_PT)],
                  num_out.at[cid, pl.ds(row0, ROWS_PT)])


def _edge_phase(src_r, dst_r, alpha_src, alpha_dst, xw):
  mesh = plsc.VectorSubcoreMesh(core_axis_name="c", subcore_axis_name="s")
  kernel_fn = functools.partial(
      pl.kernel,
      out_type=[
          jax.ShapeDtypeStruct((NC, N, HID), _f32),
          jax.ShapeDtypeStruct((NW, N), _f32),
      ],
      mesh=mesh,
      scratch_types=[
          pltpu.VMEM((N,), _f32),
          pltpu.VMEM((N,), _f32),
          pltpu.VMEM((N,), _f32),
          pltpu.VMEM((NBLK, BLK), jnp.int32),
          pltpu.VMEM((NBLK, BLK), jnp.int32),
          pltpu.VMEM((BLK, HID), _f32),
          pltpu.VMEM((BLK,), _f32),
          pltpu.VMEM_SHARED((N, HID), _f32),
          pltpu.SemaphoreType.DMA,
          pltpu.SemaphoreType.DMA,
      ],
  )(_edge_body)
  return kernel_fn(src_r, dst_r, alpha_src, alpha_dst, xw)


def kernel(h, edge_index, W1, a_src1, a_dst1, b1, W2, a_src2, a_dst2, b2,
           W_out, b_out):
  src = edge_index[0].astype(jnp.int32)
  dst = edge_index[1].astype(jnp.int32)
  pad = jnp.zeros((E_PAD - E,), jnp.int32)
  src_r = jnp.concatenate([src, pad]).reshape(NW, NBLK, BLK)
  dst_r = jnp.concatenate([dst, pad]).reshape(NW, NBLK, BLK)

  asrc1 = a_src1.reshape(HID, 1)
  adst1 = a_dst1.reshape(HID, 1)
  asrc2 = a_src2.reshape(HID, 1)
  adst2 = a_dst2.reshape(HID, 1)

  xw1, as1, ad1 = _dense_in(h, W1, asrc1, adst1)
  num1, den1 = _edge_phase(src_r, dst_r, as1.reshape(N), ad1.reshape(N), xw1)
  xw2, as2, ad2 = _dense_mid(num1, den1, b1.reshape(1, HID),
                             W2, asrc2, adst2)
  num2, den2 = _edge_phase(src_r, dst_r, as2.reshape(N), ad2.reshape(N), xw2)
  return _dense_out(num2, den2, b2.reshape(1, HID), W_out, b_out.reshape(1, 1))


# speedup vs baseline: 58.9906x; 58.9906x over previous
import functools

import jax
import jax.numpy as jnp
from jax import lax
from jax.experimental import pallas as pl
from jax.experimental.pallas import tpu as pltpu
from jax.experimental.pallas import tpu_sc as plsc

N = 10000
E = 320000
D_IN = 128
HID = 16

NC = 2
NS = 16
NW = NC * NS
L = 16

BLK = 128
NBLK = 80
EPT = NBLK * BLK
E_PAD = NW * EPT
ROWS_PT = 640
ROWS_LAST = N - (NS - 1) * ROWS_PT

_f32 = jnp.float32


def _dense_in_body(x_ref, w_ref, asrc_ref, adst_ref, xw_out, as_out, ad_out):
  xw = jnp.dot(x_ref[:], w_ref[:], preferred_element_type=_f32)
  xw_out[:] = xw
  as_out[:] = jnp.dot(xw, asrc_ref[:], preferred_element_type=_f32)
  ad_out[:] = jnp.dot(xw, adst_ref[:], preferred_element_type=_f32)


def _dense_in(x, w, asrc, adst):
  return pl.pallas_call(
      _dense_in_body,
      out_shape=[
          jax.ShapeDtypeStruct((N, HID), _f32),
          jax.ShapeDtypeStruct((N, 1), _f32),
          jax.ShapeDtypeStruct((N, 1), _f32),
      ],
  )(x, w, asrc, adst)


def _dense_mid_body(nump_ref, denp_ref, b_ref, w_ref, asrc_ref, adst_ref,
                    xw_out, as_out, ad_out):
  num = jnp.sum(nump_ref[:], axis=0)
  den = jnp.sum(denp_ref[:], axis=0)
  x = jnp.maximum(num / (den[:, None] + 1e-16) + b_ref[:], 0.0)
  xw = jnp.dot(x, w_ref[:], preferred_element_type=_f32)
  xw_out[:] = xw
  as_out[:] = jnp.dot(xw, asrc_ref[:], preferred_element_type=_f32)
  ad_out[:] = jnp.dot(xw, adst_ref[:], preferred_element_type=_f32)


def _dense_mid(nump, denp, b, w, asrc, adst):
  return pl.pallas_call(
      _dense_mid_body,
      out_shape=[
          jax.ShapeDtypeStruct((N, HID), _f32),
          jax.ShapeDtypeStruct((N, 1), _f32),
          jax.ShapeDtypeStruct((N, 1), _f32),
      ],
  )(nump, denp, b, w, asrc, adst)


def _dense_out_body(nump_ref, denp_ref, b_ref, w_ref, bout_ref, out_ref):
  num = jnp.sum(nump_ref[:], axis=0)
  den = jnp.sum(denp_ref[:], axis=0)
  x = jnp.maximum(num / (den[:, None] + 1e-16) + b_ref[:], 0.0)
  out_ref[:] = jnp.dot(x, w_ref[:], preferred_element_type=_f32) + bout_ref[:]


def _dense_out(nump, denp, b, w, bout):
  return pl.pallas_call(
      _dense_out_body,
      out_shape=jax.ShapeDtypeStruct((N, 1), _f32),
  )(nump, denp, b, w, bout)


def _edge_body(src_hbm, dst_hbm, asrc_hbm, adst_hbm, xw_hbm, zeros_hbm,
               num_out, den_out,
               asrc_v, adst_v, den_v, src_v, dst_v, rows_v, ex_v,
               num_sh, gsem, ssem):
  cid = lax.axis_index("c")
  sid = lax.axis_index("s")
  wid = sid * NC + cid

  pltpu.sync_copy(src_hbm.at[wid], src_v)
  pltpu.sync_copy(dst_hbm.at[wid], dst_v)
  pltpu.sync_copy(asrc_hbm, asrc_v)
  pltpu.sync_copy(adst_hbm, adst_v)

  zero16 = jnp.zeros((L,), _f32)

  def _zden(i, carry):
    den_v[pl.ds(i * L, L)] = zero16
    return carry
  lax.fori_loop(0, N // L, _zden, 0)

  row0 = pl.multiple_of(sid * ROWS_PT, ROWS_PT)
  last = sid == NS - 1

  @pl.when(jnp.logical_not(last))
  def _():
    pltpu.sync_copy(zeros_hbm.at[pl.ds(row0, ROWS_PT)],
                    num_sh.at[pl.ds(row0, ROWS_PT)])

  @pl.when(last)
  def _():
    pltpu.sync_copy(zeros_hbm.at[pl.ds(row0, ROWS_LAST)],
                    num_sh.at[pl.ds(row0, ROWS_LAST)])

  plsc.subcore_barrier()

  iota = lax.iota(jnp.int32, L)
  ebase = wid * EPT

  def _block(blk, carry):
    gather = pltpu.async_copy(xw_hbm.at[src_v.at[blk]], rows_v, gsem)

    def _chunk(j, c2):
      s_vec = src_v[blk, pl.ds(j * L, L)]
      d_vec = dst_v[blk, pl.ds(j * L, L)]
      a = plsc.load_gather(asrc_v, [s_vec]) + plsc.load_gather(adst_v, [d_vec])
      a = jnp.maximum(a, a * 0.2)
      ex = jnp.exp(a)
      eid = ebase + blk * BLK + j * L + iota
      ex = jnp.where(eid < E, ex, 0.0)
      plsc.addupdate_scatter(den_v, [d_vec], ex)
      ex_v[pl.ds(j * L, L)] = ex
      return c2
    lax.fori_loop(0, BLK // L, _chunk, 0)

    gather.wait()

    def _scale(j, c2):
      exv = ex_v[pl.ds(j * L, L)]
      for k in range(L):
        r = j * L + k
        rows_v[r] = rows_v[r] * exv[k]
      return c2
    lax.fori_loop(0, BLK // L, _scale, 0)

    pltpu.async_copy(rows_v, num_sh.at[dst_v.at[blk]], ssem, add=True).wait()
    return carry

  lax.fori_loop(0, NBLK, _block, 0)
  plsc.subcore_barrier()

  den0 = pl.multiple_of(wid * N, 8)
  pltpu.sync_copy(den_v, den_out.at[pl.ds(den0, N)])

  @pl.when(jnp.logical_not(last))
  def _():
    pltpu.sync_copy(num_sh.at[pl.ds(row0, ROWS_PT)],
                    num_out.at[cid, pl.ds(row0, ROWS_PT)])

  @pl.when(last)
  def _():
    pltpu.sync_copy(num_sh.at[pl.ds(row0, ROWS_LAST)],
                    num_out.at[cid, pl.ds(row0, ROWS_LAST)])


def _edge_phase(src_r, dst_r, alpha_src, alpha_dst, xw, zeros):
  mesh = plsc.VectorSubcoreMesh(core_axis_name="c", subcore_axis_name="s")
  kernel_fn = functools.partial(
      pl.kernel,
      out_type=[
          jax.ShapeDtypeStruct((NC, N, HID), _f32),
          jax.ShapeDtypeStruct((NW * N,), _f32),
      ],
      mesh=mesh,
      scratch_types=[
          pltpu.VMEM((N,), _f32),
          pltpu.VMEM((N,), _f32),
          pltpu.VMEM((N,), _f32),
          pltpu.VMEM((NBLK, BLK), jnp.int32),
          pltpu.VMEM((NBLK, BLK), jnp.int32),
          pltpu.VMEM((BLK, HID), _f32),
          pltpu.VMEM((BLK,), _f32),
          pltpu.VMEM_SHARED((N, HID), _f32),
          pltpu.SemaphoreType.DMA,
          pltpu.SemaphoreType.DMA,
      ],
      compiler_params=pltpu.CompilerParams(needs_layout_passes=False,
                                           use_tc_tiling_on_sc=False),
  )(_edge_body)
  num, den = kernel_fn(src_r, dst_r, alpha_src, alpha_dst, xw, zeros)
  return num, den.reshape(NW, N)


def kernel(h, edge_index, W1, a_src1, a_dst1, b1, W2, a_src2, a_dst2, b2,
           W_out, b_out):
  src = edge_index[0].astype(jnp.int32)
  dst = edge_index[1].astype(jnp.int32)
  pad = jnp.zeros((E_PAD - E,), jnp.int32)
  src_r = jnp.concatenate([src, pad]).reshape(NW, NBLK, BLK)
  dst_r = jnp.concatenate([dst, pad]).reshape(NW, NBLK, BLK)

  asrc1 = a_src1.reshape(HID, 1)
  adst1 = a_dst1.reshape(HID, 1)
  asrc2 = a_src2.reshape(HID, 1)
  adst2 = a_dst2.reshape(HID, 1)

  zeros = jnp.zeros((N, HID), _f32)

  xw1, as1, ad1 = _dense_in(h, W1, asrc1, adst1)
  num1, den1 = _edge_phase(src_r, dst_r, as1.reshape(N), ad1.reshape(N), xw1,
                           zeros)
  xw2, as2, ad2 = _dense_mid(num1, den1, b1.reshape(1, HID),
                             W2, asrc2, adst2)
  num2, den2 = _edge_phase(src_r, dst_r, as2.reshape(N), ad2.reshape(N), xw2,
                           zeros)
  return _dense_out(num2, den2, b2.reshape(1, HID), W_out, b_out.reshape(1, 1))
